# Initial kernel scaffold; baseline (speedup 1.0000x reference)
import jax
import jax.numpy as jnp
from jax.experimental import pallas as pl


def kernel(x, edge_index, W, b):
    raise NotImplementedError("write your pallas kernel here")


# speedup vs baseline: 7.7878x; 7.7878x over previous
import functools

import jax
import jax.numpy as jnp
from jax import lax
from jax.experimental import pallas as pl
from jax.experimental.pallas import tpu as pltpu
from jax.experimental.pallas import tpu_sc as plsc

N_NODES = 10000
NFEAT = 128
NP = 10240
NC = 2
NS = 16
NW = NC * NS
CB = 128
KC = 80
EPW = KC * CB
EP = NW * EPW
RPT = NP // NS

_MESH = plsc.VectorSubcoreMesh(
    core_axis_name="c", subcore_axis_name="s", num_cores=NC, num_subcores=NS
)


def _zero_fill(zbuf, nrows):
    ncols = zbuf.shape[1]
    z = jnp.zeros((16,), jnp.float32)
    for r in range(nrows):
        for c0 in range(ncols // 16):
            zbuf[r, pl.ds(c0 * 16, 16)] = z


@functools.partial(
    pl.kernel,
    out_type=jax.ShapeDtypeStruct((NC, NP, 16), jnp.float32),
    mesh=_MESH,
    scratch_types=[
        pltpu.VMEM((KC, CB), jnp.int32),
        pltpu.VMEM((CB, 16), jnp.float32),
        pltpu.VMEM((16, 16), jnp.float32),
        pltpu.VMEM_SHARED((NP, 16), jnp.float32),
    ],
)
def _deg_kernel(dst_hbm, cnt_hbm, dst_v, ones_v, zb_v, acc_sh):
    cid = lax.axis_index("c")
    sid = lax.axis_index("s")
    wid = sid * NC + cid
    pltpu.sync_copy(dst_hbm.at[wid], dst_v)
    one = jnp.ones((16,), jnp.float32)
    for r in range(CB):
        ones_v[r, :] = one
    _zero_fill(zb_v, 16)
    row0 = sid * RPT
    for z in range(RPT // 16):
        pltpu.sync_copy(zb_v, acc_sh.at[pl.ds(row0 + z * 16, 16)])
    plsc.subcore_barrier()

    def body(c, carry):
        pltpu.sync_copy(ones_v, acc_sh.at[dst_v.at[c]], add=True)
        return carry

    lax.fori_loop(0, KC, body, 0)
    plsc.subcore_barrier()
    pltpu.sync_copy(acc_sh.at[pl.ds(row0, RPT)], cnt_hbm.at[cid, pl.ds(row0, RPT)])


@functools.partial(
    pl.kernel,
    out_type=jax.ShapeDtypeStruct((NC, NP, NFEAT), jnp.float32),
    mesh=_MESH,
    scratch_types=[
        pltpu.VMEM((KC, CB), jnp.int32),
        pltpu.VMEM((KC, CB), jnp.int32),
        pltpu.VMEM((CB, NFEAT), jnp.float32),
        pltpu.VMEM((16, NFEAT), jnp.float32),
        pltpu.VMEM_SHARED((NP, NFEAT), jnp.float32),
        pltpu.SemaphoreType.DMA,
    ],
)
def _hop_kernel(g_hbm, src_hbm, dst_hbm, out_hbm,
                src_v, dst_v, buf0, zb_v, acc_sh, sem0):
    cid = lax.axis_index("c")
    sid = lax.axis_index("s")
    wid = sid * NC + cid
    pltpu.sync_copy(src_hbm.at[wid], src_v)
    pltpu.sync_copy(dst_hbm.at[wid], dst_v)
    _zero_fill(zb_v, 16)
    row0 = sid * RPT
    for z in range(RPT // 16):
        pltpu.sync_copy(zb_v, acc_sh.at[pl.ds(row0 + z * 16, 16)])
    plsc.subcore_barrier()

    def body(c, carry):
        pltpu.async_copy(g_hbm.at[src_v.at[c]], buf0, sem0).wait()
        pltpu.sync_copy(buf0, acc_sh.at[dst_v.at[c]], add=True)
        return carry

    lax.fori_loop(0, KC, body, 0)
    plsc.subcore_barrier()
    pltpu.sync_copy(acc_sh.at[pl.ds(row0, RPT)], out_hbm.at[cid, pl.ds(row0, RPT)])


_BR = 1024


def _dis_from_cnt(cnt_ref):
    deg = 1.0 + cnt_ref[0][:, :1] + cnt_ref[1][:, :1]
    return 1.0 / jnp.sqrt(deg)


def _prep_body(cnt_ref, x_ref, g0_ref):
    g0_ref[...] = x_ref[...] * _dis_from_cnt(cnt_ref)


def _mid_body(cnt_ref, acc_ref, g0_ref, g1_ref):
    dis = _dis_from_cnt(cnt_ref)
    g1_ref[...] = (acc_ref[0] + acc_ref[1] + g0_ref[...]) * (dis * dis)


def _out_body(cnt_ref, acc_ref, g1_ref, w_ref, b_ref, o_ref):
    h = (acc_ref[0] + acc_ref[1] + g1_ref[...]) * _dis_from_cnt(cnt_ref)
    o_ref[...] = (
        jnp.dot(h, w_ref[...], preferred_element_type=jnp.float32) + b_ref[...]
    )


_CNT_SPEC = pl.BlockSpec((NC, _BR, 16), lambda i: (0, i, 0))
_ROW_SPEC = pl.BlockSpec((_BR, NFEAT), lambda i: (i, 0))
_ACC_SPEC = pl.BlockSpec((NC, _BR, NFEAT), lambda i: (0, i, 0))

_prep = pl.pallas_call(
    _prep_body,
    grid=(NP // _BR,),
    in_specs=[_CNT_SPEC, _ROW_SPEC],
    out_specs=_ROW_SPEC,
    out_shape=jax.ShapeDtypeStruct((NP, NFEAT), jnp.float32),
)

_mid = pl.pallas_call(
    _mid_body,
    grid=(NP // _BR,),
    in_specs=[_CNT_SPEC, _ACC_SPEC, _ROW_SPEC],
    out_specs=_ROW_SPEC,
    out_shape=jax.ShapeDtypeStruct((NP, NFEAT), jnp.float32),
)

_outk = pl.pallas_call(
    _out_body,
    grid=(NP // _BR,),
    in_specs=[
        _CNT_SPEC,
        _ACC_SPEC,
        _ROW_SPEC,
        pl.BlockSpec((NFEAT, NFEAT), lambda i: (0, 0)),
        pl.BlockSpec((1, NFEAT), lambda i: (0, 0)),
    ],
    out_specs=_ROW_SPEC,
    out_shape=jax.ShapeDtypeStruct((NP, NFEAT), jnp.float32),
)


def kernel(x, edge_index, W, b):
    ei = edge_index.astype(jnp.int32)
    e = ei.shape[1]
    pad = jnp.full((2, EP - e), NP - 1, jnp.int32)
    eip = jnp.concatenate([ei, pad], axis=1)
    src_p = eip[0].reshape(NW, KC, CB)
    dst_p = eip[1].reshape(NW, KC, CB)
    x_pad = jnp.concatenate(
        [x, jnp.zeros((NP - N_NODES, NFEAT), jnp.float32)], axis=0
    )

    cnt = _deg_kernel(dst_p)
    g0 = _prep(cnt, x_pad)
    t0 = _hop_kernel(g0, src_p, dst_p)
    g1 = _mid(cnt, t0, g0)
    t1 = _hop_kernel(g1, src_p, dst_p)
    out = _outk(cnt, t1, g1, W, b.reshape(1, NFEAT))
    return out[:N_NODES]


# speedup vs baseline: 7.9450x; 1.0202x over previous
import functools

import jax
import jax.numpy as jnp
from jax import lax
from jax.experimental import pallas as pl
from jax.experimental.pallas import tpu as pltpu
from jax.experimental.pallas import tpu_sc as plsc

N_NODES = 10000
NFEAT = 128
NP = 10240
NC = 2
NS = 16
NW = NC * NS
CB = 128
KC = 80
EPW = KC * CB
EP = NW * EPW
RPT = NP // NS

_MESH = plsc.VectorSubcoreMesh(
    core_axis_name="c", subcore_axis_name="s", num_cores=NC, num_subcores=NS
)


def _zero_fill(zbuf, nrows):
    ncols = zbuf.shape[1]
    z = jnp.zeros((16,), jnp.float32)
    for r in range(nrows):
        for c0 in range(ncols // 16):
            zbuf[r, pl.ds(c0 * 16, 16)] = z


@functools.partial(
    pl.kernel,
    out_type=jax.ShapeDtypeStruct((NC, NP, 16), jnp.float32),
    mesh=_MESH,
    scratch_types=[
        pltpu.VMEM((KC, CB), jnp.int32),
        pltpu.VMEM((CB, 16), jnp.float32),
        pltpu.VMEM((16, 16), jnp.float32),
        pltpu.VMEM_SHARED((NP, 16), jnp.float32),
    ],
)
def _deg_kernel(dst_hbm, cnt_hbm, dst_v, ones_v, zb_v, acc_sh):
    cid = lax.axis_index("c")
    sid = lax.axis_index("s")
    wid = sid * NC + cid
    pltpu.sync_copy(dst_hbm.at[wid], dst_v)
    one = jnp.ones((16,), jnp.float32)
    for r in range(CB):
        ones_v[r, :] = one
    _zero_fill(zb_v, 16)
    row0 = sid * RPT
    for z in range(RPT // 16):
        pltpu.sync_copy(zb_v, acc_sh.at[pl.ds(row0 + z * 16, 16)])
    plsc.subcore_barrier()

    def body(c, carry):
        pltpu.sync_copy(ones_v, acc_sh.at[dst_v.at[c]], add=True)
        return carry

    lax.fori_loop(0, KC, body, 0)
    plsc.subcore_barrier()
    pltpu.sync_copy(acc_sh.at[pl.ds(row0, RPT)], cnt_hbm.at[cid, pl.ds(row0, RPT)])


@functools.partial(
    pl.kernel,
    out_type=jax.ShapeDtypeStruct((NC, NP, NFEAT), jnp.float32),
    mesh=_MESH,
    scratch_types=[
        pltpu.VMEM((KC // 2, CB), jnp.int32),
        pltpu.VMEM((KC // 2, CB), jnp.int32),
        pltpu.VMEM((CB, NFEAT), jnp.float32),
        pltpu.VMEM((CB, NFEAT), jnp.float32),
        pltpu.VMEM((16, NFEAT), jnp.float32),
        pltpu.VMEM_SHARED((NP, NFEAT), jnp.float32),
        pltpu.SemaphoreType.DMA,
        pltpu.SemaphoreType.DMA,
    ],
)
def _hop_kernel(g_hbm, src_hbm, dst_hbm, out_hbm,
                src_v, dst_v, buf_a, buf_b, zb_v, acc_sh, sem_a, sem_b):
    cid = lax.axis_index("c")
    sid = lax.axis_index("s")
    wid = sid * NC + cid
    _zero_fill(zb_v, 16)
    row0 = sid * RPT
    for z in range(RPT // 16):
        pltpu.sync_copy(zb_v, acc_sh.at[pl.ds(row0 + z * 16, 16)])
    plsc.subcore_barrier()

    wc = KC // 2

    for w in range(2):
        pltpu.sync_copy(src_hbm.at[wid, pl.ds(w * wc, wc)], src_v)
        pltpu.sync_copy(dst_hbm.at[wid, pl.ds(w * wc, wc)], dst_v)

        def body(t, carry):
            c0 = 2 * t
            c1 = c0 + 1
            d0 = pltpu.async_copy(g_hbm.at[src_v.at[c0]], buf_a, sem_a)
            d1 = pltpu.async_copy(g_hbm.at[src_v.at[c1]], buf_b, sem_a)
            d0.wait()
            d1.wait()
            pltpu.sync_copy(buf_a, acc_sh.at[dst_v.at[c0]], add=True)
            pltpu.sync_copy(buf_b, acc_sh.at[dst_v.at[c1]], add=True)
            return carry

        lax.fori_loop(0, wc // 2, body, 0)
    plsc.subcore_barrier()
    pltpu.sync_copy(acc_sh.at[pl.ds(row0, RPT)], out_hbm.at[cid, pl.ds(row0, RPT)])


_BR = 1024


def _dis_from_cnt(cnt_ref):
    deg = 1.0 + cnt_ref[0][:, :1] + cnt_ref[1][:, :1]
    return 1.0 / jnp.sqrt(deg)


def _prep_body(cnt_ref, x_ref, g0_ref):
    g0_ref[...] = x_ref[...] * _dis_from_cnt(cnt_ref)


def _mid_body(cnt_ref, acc_ref, g0_ref, g1_ref):
    dis = _dis_from_cnt(cnt_ref)
    g1_ref[...] = (acc_ref[0] + acc_ref[1] + g0_ref[...]) * (dis * dis)


def _out_body(cnt_ref, acc_ref, g1_ref, w_ref, b_ref, o_ref):
    h = (acc_ref[0] + acc_ref[1] + g1_ref[...]) * _dis_from_cnt(cnt_ref)
    o_ref[...] = (
        jnp.dot(h, w_ref[...], preferred_element_type=jnp.float32) + b_ref[...]
    )


_CNT_SPEC = pl.BlockSpec((NC, _BR, 16), lambda i: (0, i, 0))
_ROW_SPEC = pl.BlockSpec((_BR, NFEAT), lambda i: (i, 0))
_ACC_SPEC = pl.BlockSpec((NC, _BR, NFEAT), lambda i: (0, i, 0))

_prep = pl.pallas_call(
    _prep_body,
    grid=(NP // _BR,),
    in_specs=[_CNT_SPEC, _ROW_SPEC],
    out_specs=_ROW_SPEC,
    out_shape=jax.ShapeDtypeStruct((NP, NFEAT), jnp.float32),
)

_mid = pl.pallas_call(
    _mid_body,
    grid=(NP // _BR,),
    in_specs=[_CNT_SPEC, _ACC_SPEC, _ROW_SPEC],
    out_specs=_ROW_SPEC,
    out_shape=jax.ShapeDtypeStruct((NP, NFEAT), jnp.float32),
)

_outk = pl.pallas_call(
    _out_body,
    grid=(NP // _BR,),
    in_specs=[
        _CNT_SPEC,
        _ACC_SPEC,
        _ROW_SPEC,
        pl.BlockSpec((NFEAT, NFEAT), lambda i: (0, 0)),
        pl.BlockSpec((1, NFEAT), lambda i: (0, 0)),
    ],
    out_specs=_ROW_SPEC,
    out_shape=jax.ShapeDtypeStruct((NP, NFEAT), jnp.float32),
)


def kernel(x, edge_index, W, b):
    ei = edge_index.astype(jnp.int32)
    e = ei.shape[1]
    pad = jnp.full((2, EP - e), NP - 1, jnp.int32)
    eip = jnp.concatenate([ei, pad], axis=1)
    src_p = eip[0].reshape(NW, KC, CB)
    dst_p = eip[1].reshape(NW, KC, CB)
    x_pad = jnp.concatenate(
        [x, jnp.zeros((NP - N_NODES, NFEAT), jnp.float32)], axis=0
    )

    cnt = _deg_kernel(dst_p)
    g0 = _prep(cnt, x_pad)
    t0 = _hop_kernel(g0, src_p, dst_p)
    g1 = _mid(cnt, t0, g0)
    t1 = _hop_kernel(g1, src_p, dst_p)
    out = _outk(cnt, t1, g1, W, b.reshape(1, NFEAT))
    return out[:N_NODES]
